# Initial kernel scaffold; baseline (speedup 1.0000x reference)
import jax
import jax.numpy as jnp
from jax.experimental import pallas as pl


def kernel(text_embeds, audio_feats, speaker_ids, W_ih_f, W_hh_f, b_f, W_ih_b, W_hh_b, b_b, Wg, bg, Wx, Wgr, bf, Wc, bc):
    raise NotImplementedError("write your pallas kernel here")


# speedup vs baseline: 17.5799x; 17.5799x over previous
import functools

import jax
import jax.numpy as jnp
from jax import lax
from jax.experimental import pallas as pl
from jax.experimental.pallas import tpu as pltpu

_T = 2048
_B = 8
_H = 128
_WIN = 20
_HOPS = 3
_PREC = lax.Precision.HIGHEST


_C = 128
_K = _T // _C


def _lstm_body(text_f, audio_f, text_b, audio_b, wtf, waf, wtb, wab, wblk,
               bcat, hsf, hsb, gxf, gxb, hc):
    k = pl.program_id(0)

    xt_f = text_f[...].reshape(_C * _B, 512)
    xa_f = audio_f[...].reshape(_C * _B, 128)
    gxf[...] = (jnp.dot(xt_f, wtf[...], precision=_PREC)
                + jnp.dot(xa_f, waf[...], precision=_PREC)).reshape(_C, _B, 512)
    xt_b = text_b[...].reshape(_C * _B, 512)
    xa_b = audio_b[...].reshape(_C * _B, 128)
    gxb[...] = (jnp.dot(xt_b, wtb[...], precision=_PREC)
                + jnp.dot(xa_b, wab[...], precision=_PREC)).reshape(_C, _B, 512)

    @pl.when(k == 0)
    def _():
        hc[...] = jnp.zeros_like(hc)

    h0 = hc[0]
    c0 = hc[1]

    def step(s, carry):
        h, c = carry
        gx2 = jnp.concatenate([gxf[s], gxb[_C - 1 - s]], axis=1)
        g = gx2 + jnp.dot(h, wblk[...], precision=_PREC) + bcat[...]
        i2 = jnp.concatenate([g[:, 0:128], g[:, 512:640]], axis=1)
        f2 = jnp.concatenate([g[:, 128:256], g[:, 640:768]], axis=1)
        g2 = jnp.concatenate([g[:, 256:384], g[:, 768:896]], axis=1)
        o2 = jnp.concatenate([g[:, 384:512], g[:, 896:1024]], axis=1)
        i2 = jax.nn.sigmoid(i2)
        f2 = jax.nn.sigmoid(f2)
        g2 = jnp.tanh(g2)
        o2 = jax.nn.sigmoid(o2)
        c = f2 * c + i2 * g2
        h = o2 * jnp.tanh(c)
        hsf[s] = h[:, :128]
        hsb[_C - 1 - s] = h[:, 128:]
        return h, c

    h, c = lax.fori_loop(0, _C, step, (h0, c0))
    hc[0] = h
    hc[1] = c


def _lstm(text_tm, audio_tm, wtf, waf, wtb, wab, wblk, bcat):
    grid = (_K,)
    full = lambda *_: tuple(0 for _ in range(2))
    specs = [
        pl.BlockSpec((_C, _B, 512), lambda k: (k, 0, 0)),
        pl.BlockSpec((_C, _B, 128), lambda k: (k, 0, 0)),
        pl.BlockSpec((_C, _B, 512), lambda k: (_K - 1 - k, 0, 0)),
        pl.BlockSpec((_C, _B, 128), lambda k: (_K - 1 - k, 0, 0)),
        pl.BlockSpec((512, 512), lambda k: (0, 0)),
        pl.BlockSpec((128, 512), lambda k: (0, 0)),
        pl.BlockSpec((512, 512), lambda k: (0, 0)),
        pl.BlockSpec((128, 512), lambda k: (0, 0)),
        pl.BlockSpec((256, 1024), lambda k: (0, 0)),
        pl.BlockSpec((1, 1024), lambda k: (0, 0)),
    ]
    out_specs = [
        pl.BlockSpec((_C, _B, 128), lambda k: (k, 0, 0)),
        pl.BlockSpec((_C, _B, 128), lambda k: (_K - 1 - k, 0, 0)),
    ]
    return pl.pallas_call(
        _lstm_body,
        grid=grid,
        in_specs=specs,
        out_specs=out_specs,
        out_shape=[
            jax.ShapeDtypeStruct((_T, _B, 128), jnp.float32),
            jax.ShapeDtypeStruct((_T, _B, 128), jnp.float32),
        ],
        scratch_shapes=[
            pltpu.VMEM((_C, _B, 512), jnp.float32),
            pltpu.VMEM((_C, _B, 512), jnp.float32),
            pltpu.VMEM((2, _B, 256), jnp.float32),
        ],
    )(text_tm, audio_tm, text_tm, audio_tm, wtf, waf, wtb, wab, wblk, bcat)


_PAD = 32
_RT = 256


def _win21(src_ref, base):
    e0 = base - 24
    s1 = src_ref[pl.ds(e0, _RT + 24), :]
    def dbl(a, st, n):
        return a[n:] + a[:a.shape[0] - n], st + n
    s2, st2 = dbl(s1, e0, 1)
    s4, st4 = dbl(s2, st2, 2)
    s8, st8 = dbl(s4, st4, 4)
    s16, st16 = dbl(s8, st8, 8)
    def at(a, st, row0):
        i = row0 - st
        return a[i:i + _RT]
    return (at(s16, st16, base)
            + at(s4, st4, base - 16)
            + at(s1, e0, base - 20))


def _grn_body(hsf_b, hsb_b, idg, out, cur, nxt, acc):
    ntile = _T // _RT
    cur[pl.ds(0, _PAD), :] = jnp.zeros((_PAD, 256), jnp.float32)
    nxt[pl.ds(0, _PAD), :] = jnp.zeros((_PAD, 256), jnp.float32)
    for rt in range(ntile):
        r = rt * _RT
        x = jnp.concatenate([hsf_b[pl.ds(r, _RT), 0, 0, :],
                             hsb_b[pl.ds(r, _RT), 0, 0, :]], axis=1)
        cur[pl.ds(_PAD + r, _RT), :] = x
        acc[pl.ds(r, _RT), :] = x
    src, dst = cur, nxt
    for _ in range(_HOPS):
        for rt in range(ntile):
            r = rt * _RT
            w = _win21(src, _PAD + r)
            nv = w * idg[pl.ds(r, _RT), :]
            dst[pl.ds(_PAD + r, _RT), :] = nv
            acc[pl.ds(r, _RT), :] = acc[pl.ds(r, _RT), :] + nv
        src, dst = dst, src
    for rt in range(ntile):
        r = rt * _RT
        out[pl.ds(r, _RT), 0, 0, :] = acc[pl.ds(r, _RT), :] * 0.25


def _grn(hsf, hsb, idg):
    out = pl.pallas_call(
        _grn_body,
        grid=(_B,),
        in_specs=[
            pl.BlockSpec((_T, 1, 1, 128), lambda b: (0, b, 0, 0)),
            pl.BlockSpec((_T, 1, 1, 128), lambda b: (0, b, 0, 0)),
            pl.BlockSpec((_T, 1), lambda b: (0, 0)),
        ],
        out_specs=pl.BlockSpec((_T, 1, 1, 256), lambda b: (0, b, 0, 0)),
        out_shape=jax.ShapeDtypeStruct((_T, _B, 1, 256), jnp.float32),
        scratch_shapes=[
            pltpu.VMEM((_T + _PAD, 256), jnp.float32),
            pltpu.VMEM((_T + _PAD, 256), jnp.float32),
            pltpu.VMEM((_T, 256), jnp.float32),
        ],
    )(hsf.reshape(_T, _B, 1, 128), hsb.reshape(_T, _B, 1, 128), idg)
    return out.reshape(_T, _B, 256)


_CF = 256


def _fusion_body(hsf, hsb, gr, wg_l, wg_g, bg, wx, wgr, bfv, wc, bc, out):
    n = _CF * _B
    l = jnp.concatenate([hsf[...].reshape(n, 128), hsb[...].reshape(n, 128)],
                        axis=1)
    g = gr[...].reshape(n, 256)
    gate = jax.nn.sigmoid(jnp.dot(l, wg_l[...], precision=_PREC)
                          + jnp.dot(g, wg_g[...], precision=_PREC) + bg[...])
    fused = jnp.tanh(gate * jnp.dot(l, wx[...], precision=_PREC)
                     + (1.0 - gate) * jnp.dot(g, wgr[...], precision=_PREC)
                     + bfv[...])
    out[...] = (jnp.dot(fused, wc[...], precision=_PREC)
                + bc[...]).reshape(_CF, _B, 128)


def _fusion(hsf, hsb, graph, wg_l, wg_g, bg, wx, wgr, bfv, wc, bc):
    m = _T // _CF
    wspec = lambda shp: pl.BlockSpec(shp, lambda k: (0, 0))
    return pl.pallas_call(
        _fusion_body,
        grid=(m,),
        in_specs=[
            pl.BlockSpec((_CF, _B, 128), lambda k: (k, 0, 0)),
            pl.BlockSpec((_CF, _B, 128), lambda k: (k, 0, 0)),
            pl.BlockSpec((_CF, _B, 256), lambda k: (k, 0, 0)),
            wspec((256, 256)), wspec((256, 256)), wspec((1, 256)),
            wspec((256, 256)), wspec((256, 256)), wspec((1, 256)),
            wspec((256, 128)), wspec((1, 128)),
        ],
        out_specs=pl.BlockSpec((_CF, _B, 128), lambda k: (k, 0, 0)),
        out_shape=jax.ShapeDtypeStruct((_T, _B, 128), jnp.float32),
    )(hsf, hsb, graph, wg_l, wg_g, bg, wx, wgr, bfv, wc, bc)


def kernel(text_embeds, audio_feats, speaker_ids, W_ih_f, W_hh_f, b_f,
           W_ih_b, W_hh_b, b_b, Wg, bg, Wx, Wgr, bf, Wc, bc):
    del speaker_ids
    f32 = jnp.float32

    text_tm = jnp.swapaxes(text_embeds, 0, 1)
    audio_tm = jnp.swapaxes(audio_feats, 0, 1)

    wtf = W_ih_f[:, :512].T
    waf = W_ih_f[:, 512:].T
    wtb = W_ih_b[:, :512].T
    wab = W_ih_b[:, 512:].T
    wblk = jnp.zeros((256, 1024), f32)
    wblk = wblk.at[:128, :512].set(W_hh_f.T)
    wblk = wblk.at[128:, 512:].set(W_hh_b.T)
    bcat = jnp.concatenate([b_f, b_b]).reshape(1, 1024)

    hsf, hsb = _lstm(text_tm, audio_tm, wtf, waf, wtb, wab, wblk, bcat)

    idg = (1.0 / jnp.minimum(jnp.arange(_T, dtype=f32) + 1.0, 21.0))
    graph = _grn(hsf, hsb, idg.reshape(_T, 1))

    wg_l = Wg[:, :256].T
    wg_g = Wg[:, 256:].T
    wc_pad = jnp.zeros((256, 128), f32).at[:, :7].set(Wc.T)
    bc_pad = jnp.zeros((1, 128), f32).at[0, :7].set(bc)
    out = _fusion(hsf, hsb, graph, wg_l, wg_g, bg.reshape(1, 256),
                  Wx.T, Wgr.T, bf.reshape(1, 256), wc_pad, bc_pad)

    return jnp.swapaxes(out[:, :, :7], 0, 1)
